# baseline (speedup 1.0000x reference)
import jax
import jax.numpy as jnp
from jax import lax
from jax.experimental import pallas as pl
from jax.experimental.pallas import tpu as pltpu
from jax.experimental.pallas import tpu_sc as plsc

_F = 26
_V = 100000
_E = 32
_B = 16384
_TOT = _B * _F
_CC = 1024
_NK = 98
_V0 = (_NK - 1) * _CC
_NBF = 144
_CAP = _NBF * 128
_NBAT = _F * _NBF
_NW = 32
_PERW = _NBAT // _NW
_RINGS = 8
_ROUNDS = (_PERW + _RINGS - 1) // _RINGS


def _call1_body(tab_t, tab_edge, xe_p, vals_hbm, rlist_hbm,
                xe_v, counts_v, cursor_v, rows_v, cols_v, strip_v, outbuf_v):
    nc = 2
    wid = lax.axis_index("s") * nc + lax.axis_index("c")

    @pl.when(wid < _F)
    def _():
        f = wid
        iota = lax.iota(jnp.int32, 16)
        iota32 = iota * 32

        pltpu.sync_copy(xe_p.at[f], xe_v)

        zeros16 = jnp.zeros((16,), jnp.int32)
        for si in range(8):
            counts_v[pl.ds(si * 16, 16)] = zeros16

        trash16 = jnp.full((16,), _TOT, jnp.int32)

        def init_rows(j, c):
            rows_v[j >> 3, pl.ds((j & 7) * 16, 16)] = trash16
            return c

        lax.fori_loop(0, _NBF * 8, init_rows, 0)

        def pass1(j, c):
            x16 = xe_v[j >> 3, pl.ds((j & 7) * 16, 16)]
            c16 = x16 >> 10
            cnt, last = plsc.scan_count(c16)
            plsc.addupdate_scatter(counts_v, [c16], cnt, mask=last)
            return c

        lax.fori_loop(0, _B // 16, pass1, 0)

        carry = jnp.int32(0)
        for si in range(8):
            c16 = counts_v[pl.ds(si * 16, 16)]
            p16 = jnp.bitwise_and(c16 + 15, jnp.int32(-16))
            cs = plsc.cumsum(p16)
            cursor_v[pl.ds(si * 16, 16)] = cs - p16 + carry
            carry = carry + jnp.max(cs)

        def pass2(j, c):
            x16 = xe_v[j >> 3, pl.ds((j & 7) * 16, 16)]
            c16 = x16 >> 10
            cnt, last = plsc.scan_count(c16)
            cur = plsc.load_gather(cursor_v, [c16])
            pos = cur + (cnt - 1)
            r16 = (iota + j * 16) * _F + f
            plsc.store_scatter(rows_v, [pos >> 7, pos & 127], r16)
            plsc.store_scatter(cols_v, [pos], x16)
            plsc.addupdate_scatter(cursor_v, [c16], cnt, mask=last)
            return c

        lax.fori_loop(0, _B // 16, pass2, 0)

        pltpu.sync_copy(rows_v, rlist_hbm.at[f])

        comp_rows = [jnp.full((16,), comp, jnp.int32) for comp in range(_E)]

        def bucket_count(kk):
            return counts_v[pl.ds(kk, 16)][0]

        def chunk_groups(kk, col0, off, nout):
            cntk = bucket_count(kk)
            ngk = (cntk + 15) >> 4

            def g(gi, c):
                eoff = off + gi * 16
                nout_g = nout + gi * 16
                cols16 = cols_v[pl.ds(eoff, 16)]
                lc16 = jnp.clip(cols16 - col0, 0, _CC - 1)
                pb16 = iota32 + (nout_g & 127) * 32
                for comp in range(_E):
                    v = plsc.load_gather(strip_v, [comp_rows[comp], lc16])
                    p16 = pb16 + comp
                    plsc.store_scatter(outbuf_v, [p16 >> 7, p16 & 127], v)
                new_nout = nout_g + 16

                @pl.when((new_nout & 127) == 0)
                def _flush():
                    q = (new_nout >> 7) - 1
                    pltpu.sync_copy(outbuf_v,
                                    vals_hbm.at[f, pl.ds(q * 32, 32), :])

                return c

            lax.fori_loop(0, ngk, g, 0)
            return off + ngk * 16, nout + ngk * 16

        def kloop(kk, carry):
            off, nout = carry
            cntk = bucket_count(kk)

            @pl.when(cntk > 0)
            def _load():
                for e8 in range(4):
                    pltpu.sync_copy(
                        tab_t.at[f, pl.ds(e8 * 8, 8), pl.ds(kk * _CC, _CC)],
                        strip_v.at[pl.ds(e8 * 8, 8), :])

            return chunk_groups(kk, kk * _CC, off, nout)

        off, nout = lax.fori_loop(0, _NK - 1, kloop,
                                  (jnp.int32(0), jnp.int32(0)))

        cnt_t = bucket_count(_NK - 1)

        @pl.when(cnt_t > 0)
        def _load_tail():
            for e8 in range(4):
                pltpu.sync_copy(tab_edge.at[f, pl.ds(e8 * 8, 8), :],
                                strip_v.at[pl.ds(e8 * 8, 8), :])

        off, nout = chunk_groups(_NK - 1, _V0, off, nout)

        @pl.when((nout & 127) != 0)
        def _final_flush():
            q = nout >> 7
            pltpu.sync_copy(outbuf_v, vals_hbm.at[f, pl.ds(q * 32, 32), :])


def _call2_body(vals_hbm, rl_hbm, out_hbm, buf_v, idx_v, lsem, ssem):
    nc = 2
    wid = lax.axis_index("s") * nc + lax.axis_index("c")
    start = wid * _PERW

    def load_start(t, slot):
        pltpu.async_copy(vals_hbm.at[pl.ds((start + t) * 128, 128)],
                         buf_v.at[slot], lsem.at[slot])
        pltpu.async_copy(rl_hbm.at[pl.ds(start + t, 1)],
                         idx_v.at[pl.ds(slot, 1)], lsem.at[slot])

    def load_wait(slot):
        pltpu.make_async_copy(vals_hbm.at[pl.ds(0, 128)],
                              buf_v.at[slot], lsem.at[slot]).wait()
        pltpu.make_async_copy(rl_hbm.at[pl.ds(0, 1)],
                              idx_v.at[pl.ds(slot, 1)], lsem.at[slot]).wait()

    def scat_start(slot):
        pltpu.async_copy(buf_v.at[slot], out_hbm.at[idx_v.at[slot]],
                         ssem.at[slot])

    def scat_wait(slot):
        pltpu.make_async_copy(buf_v.at[slot], out_hbm.at[idx_v.at[slot]],
                              ssem.at[slot]).wait()

    for slot in range(_RINGS):
        load_start(slot, slot)

    def round_fn(r, c):
        for b in range(_RINGS):
            t = r * _RINGS + b

            @pl.when(t < _PERW)
            def _():
                load_wait(b)
                scat_start(b)
        for b in range(_RINGS):
            t2 = (r + 1) * _RINGS + b

            @pl.when(t2 < _PERW)
            def _():
                scat_wait(b)
                load_start(t2, b)
        return c

    lax.fori_loop(0, _ROUNDS, round_fn, 0)

    for b in range(_RINGS):
        scat_wait(b)


def kernel(xe, tables):
    mesh = plsc.VectorSubcoreMesh(core_axis_name="c", subcore_axis_name="s")

    tab_t = jnp.transpose(tables, (0, 2, 1))
    tab_edge = jnp.pad(tab_t[:, :, _V0:], ((0, 0), (0, 0), (0, _CC - (_V - _V0))))
    xe_p = xe.T.reshape(_F, 128, 128)

    call1 = pl.kernel(
        _call1_body,
        out_type=(jax.ShapeDtypeStruct((_F, _NBF * 32, 128), jnp.float32),
                  jax.ShapeDtypeStruct((_F, _NBF, 128), jnp.int32)),
        mesh=mesh,
        scratch_types=[
            pltpu.VMEM((128, 128), jnp.int32),
            pltpu.VMEM((128,), jnp.int32),
            pltpu.VMEM((128,), jnp.int32),
            pltpu.VMEM((_NBF, 128), jnp.int32),
            pltpu.VMEM((_CAP,), jnp.int32),
            pltpu.VMEM((_E, _CC), jnp.float32),
            pltpu.VMEM((32, 128), jnp.float32),
        ],
        compiler_params=pltpu.CompilerParams(use_tc_tiling_on_sc=True,
                                             needs_layout_passes=False),
    )
    vals, rlist = call1(tab_t, tab_edge, xe_p)

    vflat = vals.reshape(_NBAT * 128, _E)
    rflat = rlist.reshape(_NBAT, 128)

    call2 = pl.kernel(
        _call2_body,
        out_type=jax.ShapeDtypeStruct((_TOT + 8, _E), jnp.float32),
        mesh=mesh,
        scratch_types=[
            pltpu.VMEM((_RINGS, 128, _E), jnp.float32),
            pltpu.VMEM((_RINGS, 128), jnp.int32),
            pltpu.SemaphoreType.DMA((_RINGS,)),
            pltpu.SemaphoreType.DMA((_RINGS,)),
        ],
        compiler_params=pltpu.CompilerParams(use_tc_tiling_on_sc=False),
    )
    out = call2(vflat, rflat)
    return out[:_TOT].reshape(_B, _F * _E)


# speedup vs baseline: 3.2391x; 3.2391x over previous
import jax
import jax.numpy as jnp
from jax import lax
from jax.experimental import pallas as pl
from jax.experimental.pallas import tpu as pltpu
from jax.experimental.pallas import tpu_sc as plsc

_F = 26
_V = 100000
_E = 32
_B = 16384
_TOT = _B * _F
_CC = 1024
_NK = 98
_V0 = (_NK - 1) * _CC
_NBF = 144
_CAP = _NBF * 128
_NW = 32
_PERW = _TOT // _NW
_NCH2 = _PERW // 128
_RINGS = 8


def _call1_body(tab_t, tab_edge, xe_p, vals_hbm, inv_hbm,
                xe_v, counts_v, cursor_v, inv_v, cols_v, strip2, outbuf2,
                ssem2, fsem):
    nc = 2
    wid = lax.axis_index("s") * nc + lax.axis_index("c")

    @pl.when(wid < _F)
    def _():
        f = wid
        iota = lax.iota(jnp.int32, 16)
        iota32 = iota * 32

        pltpu.sync_copy(xe_p.at[f], xe_v)

        zeros16 = jnp.zeros((16,), jnp.int32)
        for si in range(8):
            counts_v[pl.ds(si * 16, 16)] = zeros16

        def pass1(j, c):
            x16 = xe_v[j >> 3, pl.ds((j & 7) * 16, 16)]
            c16 = x16 >> 10
            cnt, last = plsc.scan_count(c16)
            plsc.addupdate_scatter(counts_v, [c16], cnt, mask=last)
            return c

        lax.fori_loop(0, _B // 16, pass1, 0)

        carry = jnp.int32(0)
        for si in range(8):
            c16 = counts_v[pl.ds(si * 16, 16)]
            p16 = jnp.bitwise_and(c16 + 15, jnp.int32(-16))
            cs = plsc.cumsum(p16)
            cursor_v[pl.ds(si * 16, 16)] = cs - p16 + carry
            carry = carry + jnp.max(cs)

        fcap = f * _CAP

        def pass2(j, c):
            x16 = xe_v[j >> 3, pl.ds((j & 7) * 16, 16)]
            c16 = x16 >> 10
            cnt, last = plsc.scan_count(c16)
            cur = plsc.load_gather(cursor_v, [c16])
            pos = cur + (cnt - 1)
            plsc.store_scatter(cols_v, [pos], x16)
            inv_v[j >> 3, pl.ds((j & 7) * 16, 16)] = pos + fcap
            plsc.addupdate_scatter(cursor_v, [c16], cnt, mask=last)
            return c

        lax.fori_loop(0, _B // 16, pass2, 0)

        pltpu.sync_copy(inv_v, inv_hbm.at[f])

        comp_rows = [jnp.full((16,), comp, jnp.int32) for comp in range(_E)]

        def bucket_count(kk):
            return counts_v[pl.ds(kk, 16)][0]

        def strip_load(src_slices, par):
            for e8 in range(4):
                pltpu.async_copy(src_slices[e8],
                                 strip2.at[par, pl.ds(e8 * 8, 8), :],
                                 ssem2.at[par])

        def strip_wait(par):
            for e8 in range(4):
                pltpu.make_async_copy(tab_edge.at[0, pl.ds(0, 8), :],
                                      strip2.at[par, pl.ds(e8 * 8, 8), :],
                                      ssem2.at[par]).wait()

        def main_slices(kk):
            return [tab_t.at[f, pl.ds(e8 * 8, 8), pl.ds(kk * _CC, _CC)]
                    for e8 in range(4)]

        def chunk_groups(kk, col0, par, off, nout):
            cntk = bucket_count(kk)
            ngk = (cntk + 15) >> 4

            def g(gi, c):
                eoff = off + gi * 16
                nout_g = nout + gi * 16
                q_cur = (nout_g >> 7) & 1
                cols16 = cols_v[pl.ds(eoff, 16)]
                lc16 = jnp.clip(cols16 - col0, 0, _CC - 1)
                pb16 = iota32 + (nout_g & 127) * 32
                ob = outbuf2.at[q_cur]
                sb = strip2.at[par]
                for comp in range(_E):
                    v = plsc.load_gather(sb, [comp_rows[comp], lc16])
                    p16 = pb16 + comp
                    plsc.store_scatter(ob, [p16 >> 7, p16 & 127], v)
                new_nout = nout_g + 16

                @pl.when((new_nout & 127) == 0)
                def _flush():
                    q = (new_nout >> 7) - 1
                    fpar = q & 1

                    @pl.when(q >= 1)
                    def _wait_other():
                        pltpu.make_async_copy(
                            outbuf2.at[1 - fpar],
                            vals_hbm.at[f, pl.ds(0, 32), :],
                            fsem.at[1 - fpar]).wait()

                    pltpu.async_copy(outbuf2.at[fpar],
                                     vals_hbm.at[f, pl.ds(q * 32, 32), :],
                                     fsem.at[fpar])

                return c

            lax.fori_loop(0, ngk, g, 0)
            return off + ngk * 16, nout + ngk * 16

        strip_load(main_slices(0), 0)

        def kloop(kk, carry):
            off, nout = carry
            par = kk & 1
            strip_wait(par)

            @pl.when(kk + 1 < _NK - 1)
            def _prefetch():
                strip_load(main_slices(kk + 1), (kk + 1) & 1)

            return chunk_groups(kk, kk * _CC, par, off, nout)

        off, nout = lax.fori_loop(0, _NK - 1, kloop,
                                  (jnp.int32(0), jnp.int32(0)))

        tail_par = (_NK - 1) & 1
        strip_load([tab_edge.at[f, pl.ds(e8 * 8, 8), :] for e8 in range(4)],
                   tail_par)
        strip_wait(tail_par)
        off, nout = chunk_groups(_NK - 1, _V0, tail_par, off, nout)

        qt = nout >> 7

        @pl.when(qt >= 1)
        def _drain_flush():
            pltpu.make_async_copy(outbuf2.at[(qt - 1) & 1],
                                  vals_hbm.at[f, pl.ds(0, 32), :],
                                  fsem.at[(qt - 1) & 1]).wait()

        @pl.when((nout & 127) != 0)
        def _final_flush():
            pltpu.sync_copy(outbuf2.at[qt & 1],
                            vals_hbm.at[f, pl.ds(qt * 32, 32), :])


def _call2_body(vals_hbm, inv_hbm, out_hbm, invb_v, idxs_v, buf_v,
                gsem, ssem):
    nc = 2
    wid = lax.axis_index("s") * nc + lax.axis_index("c")
    b0 = wid * (_B // _NW)
    row0 = wid * _NCH2

    pltpu.sync_copy(inv_hbm.at[:, pl.ds(b0, _B // _NW)], invb_v)

    iota = lax.iota(jnp.int32, 16)
    magic = jnp.int32(40330)

    def assemble(j, c):
        rl16 = iota + j * 16
        bl16 = jnp.int32((rl16 * magic) >> 20)
        f16 = rl16 - bl16 * _F
        pos16 = plsc.load_gather(invb_v, [f16, bl16])
        idxs_v[j >> 3, pl.ds((j & 7) * 16, 16)] = pos16
        return c

    lax.fori_loop(0, _PERW // 16, assemble, 0)

    def gather_start(j, b):
        pltpu.async_copy(vals_hbm.at[idxs_v.at[j]], buf_v.at[b], gsem.at[b])

    def gather_wait(b):
        pltpu.make_async_copy(vals_hbm.at[idxs_v.at[0]], buf_v.at[b],
                              gsem.at[b]).wait()

    def store_start(j, b):
        pltpu.async_copy(buf_v.at[b], out_hbm.at[pl.ds((row0 + j) * 128, 128)],
                         ssem.at[b])

    def store_wait(b):
        pltpu.make_async_copy(buf_v.at[b], out_hbm.at[pl.ds(0, 128)],
                              ssem.at[b]).wait()

    for b in range(_RINGS):
        gather_start(b, b)

    nrounds = _NCH2 // _RINGS

    def round_fn(r, c):
        j0 = r * _RINGS
        for b in range(_RINGS):
            gather_wait(b)
            store_start(j0 + b, b)
        for b in range(_RINGS):
            @pl.when(r < nrounds - 1)
            def _():
                store_wait(b)
                gather_start(j0 + _RINGS + b, b)
        return c

    lax.fori_loop(0, nrounds, round_fn, 0)

    for b in range(_RINGS):
        store_wait(b)


def kernel(xe, tables):
    mesh = plsc.VectorSubcoreMesh(core_axis_name="c", subcore_axis_name="s")

    tab_t = jnp.transpose(tables, (0, 2, 1))
    tab_edge = jnp.pad(tab_t[:, :, _V0:], ((0, 0), (0, 0), (0, _CC - (_V - _V0))))
    xe_p = xe.T.reshape(_F, 128, 128)

    call1 = pl.kernel(
        _call1_body,
        out_type=(jax.ShapeDtypeStruct((_F, _NBF * 32, 128), jnp.float32),
                  jax.ShapeDtypeStruct((_F, 128, 128), jnp.int32)),
        mesh=mesh,
        scratch_types=[
            pltpu.VMEM((128, 128), jnp.int32),
            pltpu.VMEM((128,), jnp.int32),
            pltpu.VMEM((128,), jnp.int32),
            pltpu.VMEM((128, 128), jnp.int32),
            pltpu.VMEM((_CAP,), jnp.int32),
            pltpu.VMEM((2, _E, _CC), jnp.float32),
            pltpu.VMEM((2, 32, 128), jnp.float32),
            pltpu.SemaphoreType.DMA((2,)),
            pltpu.SemaphoreType.DMA((2,)),
        ],
        compiler_params=pltpu.CompilerParams(use_tc_tiling_on_sc=True,
                                             needs_layout_passes=False),
    )
    vals, inv = call1(tab_t, tab_edge, xe_p)

    vflat = vals.reshape(_F * _NBF * 128, _E)
    iflat = inv.reshape(_F, _B)

    call2 = pl.kernel(
        _call2_body,
        out_type=jax.ShapeDtypeStruct((_TOT, _E), jnp.float32),
        mesh=mesh,
        scratch_types=[
            pltpu.VMEM((_F, _B // _NW), jnp.int32),
            pltpu.VMEM((_NCH2, 128), jnp.int32),
            pltpu.VMEM((_RINGS, 128, _E), jnp.float32),
            pltpu.SemaphoreType.DMA((_RINGS,)),
            pltpu.SemaphoreType.DMA((_RINGS,)),
        ],
        compiler_params=pltpu.CompilerParams(use_tc_tiling_on_sc=False,
                                             needs_layout_passes=False),
    )
    out = call2(vflat, iflat)
    return out.reshape(_B, _F * _E)
